# baseline (speedup 1.0000x reference)
import functools

import numpy as np
import jax
import jax.numpy as jnp
from jax import lax
from jax.experimental import pallas as pl
from jax.experimental.pallas import tpu as pltpu
from jax.experimental.pallas import tpu_sc as plsc

VOCAB = 1000000
D = 64
DP = 128
B = 4096
L = 200
MAX_LEN = 512

NC = 2
NS = 16
NW = NC * NS

BW = B // NW
NBUF = 4
PREF = 2
NOBUF = 2


def _pos_encoding(max_len, d):
    pos = np.arange(max_len, dtype=np.float32)[:, None]
    div = np.exp(np.arange(0, d, 2, dtype=np.float32) * (-np.log(10000.0) / d))
    pe = np.zeros((max_len, d), dtype=np.float32)
    pe[:, 0::2] = np.sin(pos * div)
    pe[:, 1::2] = np.cos(pos * div)
    return pe


_PE_PAD_NP = np.zeros((L, DP), dtype=np.float32)
_PE_PAD_NP[:, :D] = _pos_encoding(MAX_LEN, D)[:L]
_SCALE = float(np.sqrt(np.float32(D)))


def _pe_pad():
    return jnp.asarray(_PE_PAD_NP) + jnp.zeros((L, DP), jnp.float32)


@functools.partial(
    pl.kernel,
    out_type=jax.ShapeDtypeStruct((L, D, B), jnp.float32),
    mesh=plsc.VectorSubcoreMesh(
        core_axis_name="c", subcore_axis_name="s", num_cores=NC, num_subcores=NS
    ),
    scratch_types=[
        [pltpu.VMEM((1, BW), jnp.int32) for _ in range(NBUF)],
        [pltpu.VMEM((BW, DP), jnp.float32) for _ in range(NBUF)],
        [pltpu.VMEM((D, BW), jnp.float32) for _ in range(NOBUF)],
        pltpu.VMEM((L, DP), jnp.float32),
        [pltpu.SemaphoreType.DMA for _ in range(NBUF)],
        [pltpu.SemaphoreType.DMA for _ in range(NOBUF)],
    ],
    compiler_params=pltpu.CompilerParams(
        use_tc_tiling_on_sc=True, needs_layout_passes=False
    ),
)
def _emb_kernel(seq_t_hbm, w_hbm, pe_hbm, out_hbm,
                idx_bufs, rows_bufs, obufs, pe_v, gsems, osems):
    wid = lax.axis_index("s") * NC + lax.axis_index("c")
    b0 = wid * BW
    pltpu.sync_copy(pe_hbm, pe_v)

    iota = jax.lax.iota(jnp.int32, 16)
    bvecs = [bg * 16 + iota for bg in range(BW // 16)]

    def fire_gather(g, p):
        pltpu.sync_copy(
            seq_t_hbm.at[pl.ds(g, 1), pl.ds(b0, BW)], idx_bufs[p]
        )
        pltpu.async_copy(
            w_hbm.at[idx_bufs[p].at[0]], rows_bufs[p], gsems[p]
        )

    def wait_gather(p):
        pltpu.make_async_copy(
            w_hbm.at[pl.ds(0, BW)], rows_bufs[p], gsems[p]
        ).wait()

    def wait_out(q):
        pltpu.make_async_copy(
            obufs[q], out_hbm.at[0, :, pl.ds(0, BW)], osems[q]
        ).wait()

    for g0 in range(PREF):
        fire_gather(g0, g0)

    def outer(h, carry):
        for pp in range(NBUF):
            g = h * NBUF + pp
            q = pp % NOBUF
            p2 = (pp + PREF) % NBUF

            @pl.when(g + PREF < L)
            def _():
                fire_gather(g + PREF, p2)

            wait_gather(pp)

            @pl.when(g >= NOBUF)
            def _():
                wait_out(q)

            rows = rows_bufs[pp]
            ob = obufs[q]

            def dloop(dj, c):
                pev = pe_v[g, pl.ds(dj * 16, 16)]
                for d16 in range(16):
                    d = dj * 16 + d16
                    pe_s = pev[d16]
                    col = jnp.full((16,), d, jnp.int32)
                    for bg in range(BW // 16):
                        v = plsc.load_gather(rows, [bvecs[bg], col])
                        ob[d, pl.ds(bg * 16, 16)] = v * _SCALE + pe_s
                return c

            lax.fori_loop(0, D // 16, dloop, 0)
            pltpu.async_copy(
                ob, out_hbm.at[g, :, pl.ds(b0, BW)], osems[q]
            )
        return carry

    lax.fori_loop(0, L // NBUF, outer, 0)

    for q in range(NOBUF):
        wait_out(q)


def kernel(seq, weight):
    wpad = jnp.pad(weight, ((0, 0), (0, DP - D)))
    kout = _emb_kernel(seq.T, wpad, _pe_pad())
    return jnp.transpose(kout, (2, 0, 1))


# speedup vs baseline: 1.6934x; 1.6934x over previous
import functools

import numpy as np
import jax
import jax.numpy as jnp
from jax import lax
from jax.experimental import pallas as pl
from jax.experimental.pallas import tpu as pltpu
from jax.experimental.pallas import tpu_sc as plsc

VOCAB = 1000000
D = 64
DP = 128
B = 4096
L = 200
MAX_LEN = 512

NC = 2
NS = 16
NW = NC * NS

BW = B // NW
NBUF = 2
NOBUF = 2

NTFULL = VOCAB // DP
RING = (NTFULL + NW - 1) // NW + 1


def _pos_encoding(max_len, d):
    pos = np.arange(max_len, dtype=np.float32)[:, None]
    div = np.exp(np.arange(0, d, 2, dtype=np.float32) * (-np.log(10000.0) / d))
    pe = np.zeros((max_len, d), dtype=np.float32)
    pe[:, 0::2] = np.sin(pos * div)
    pe[:, 1::2] = np.cos(pos * div)
    return pe


_PE_PAD_NP = np.zeros((L, DP), dtype=np.float32)
_PE_PAD_NP[:, :D] = _pos_encoding(MAX_LEN, D)[:L]
_SCALE = float(np.sqrt(np.float32(D)))


def _pe_pad():
    return jnp.asarray(_PE_PAD_NP) + jnp.zeros((L, DP), jnp.float32)


def _mesh():
    return plsc.VectorSubcoreMesh(
        core_axis_name="c", subcore_axis_name="s", num_cores=NC, num_subcores=NS
    )


_PARAMS = pltpu.CompilerParams(
    use_tc_tiling_on_sc=True, needs_layout_passes=False
)


def _transpose_diag(src, dst, iota, nb, nd, scale=None, pe=None, g=None):
    def kloop(k, c):
        rot = (iota + k) & 15
        for dj in range(nd):
            dvec = rot + dj * 16
            pev = None
            if pe is not None:
                pev = plsc.load_gather(pe, [iota * 0 + g, dvec])
            for bg in range(nb):
                bvec = bg * 16 + iota
                v = plsc.load_gather(src, [bvec, dvec])
                if scale is not None:
                    v = v * scale
                if pev is not None:
                    v = v + pev
                plsc.store_scatter(dst, [dvec, bvec], v)
        return c

    lax.fori_loop(0, 16, kloop, 0)


@functools.partial(
    pl.kernel,
    out_type=jax.ShapeDtypeStruct((VOCAB, DP), jnp.float32),
    mesh=_mesh(),
    scratch_types=[
        [pltpu.VMEM((D, DP), jnp.float32) for _ in range(2)],
        [pltpu.VMEM((DP, DP), jnp.float32) for _ in range(2)],
        [pltpu.SemaphoreType.DMA for _ in range(2)],
        [pltpu.SemaphoreType.DMA for _ in range(2)],
    ],
    compiler_params=_PARAMS,
)
def _relayout_kernel(wt_hbm, tail_hbm, tab_hbm, sbufs, dbufs, isems, osems):
    wid = lax.axis_index("s") * NC + lax.axis_index("c")
    iota = jax.lax.iota(jnp.int32, 16)

    @pl.when(wid == NW - 1)
    def _():
        pltpu.sync_copy(tail_hbm, sbufs[0])
        _transpose_diag(sbufs[0], dbufs[0], iota, D // 16, DP // 16,
                        scale=_SCALE)
        pltpu.sync_copy(
            dbufs[0].at[pl.ds(0, VOCAB - NTFULL * DP)],
            tab_hbm.at[pl.ds(NTFULL * DP, VOCAB - NTFULL * DP)],
        )

    def fire_in(it, p):
        tc = it * NW + wid

        @pl.when(tc < NTFULL)
        def _():
            pltpu.async_copy(
                wt_hbm.at[:, pl.ds(tc * DP, DP)], sbufs[p], isems[p]
            )

    def outer(h, carry):
        for e in range(2):
            it = h * 2 + e
            tc = it * NW + wid
            fire_in(it + 1, (e + 1) % 2)

            @pl.when(tc < NTFULL)
            def _():
                pltpu.make_async_copy(
                    wt_hbm.at[:, pl.ds(0, DP)], sbufs[e], isems[e]
                ).wait()

                @pl.when(it >= 2)
                def _():
                    pltpu.make_async_copy(
                        dbufs[e], tab_hbm.at[pl.ds(0, DP)], osems[e]
                    ).wait()

                _transpose_diag(sbufs[e], dbufs[e], iota, D // 16, DP // 16,
                                scale=_SCALE)
                pltpu.async_copy(
                    dbufs[e], tab_hbm.at[pl.ds(tc * DP, DP)], osems[e]
                )
        return carry

    fire_in(0, 0)
    lax.fori_loop(0, RING // 2, outer, 0)

    for e in range(2):
        pltpu.make_async_copy(
            dbufs[e], tab_hbm.at[pl.ds(0, DP)], osems[e]
        ).wait()


@functools.partial(
    pl.kernel,
    out_type=jax.ShapeDtypeStruct((L, D, B), jnp.float32),
    mesh=_mesh(),
    scratch_types=[
        [pltpu.VMEM((1, BW), jnp.int32) for _ in range(NBUF)],
        [pltpu.VMEM((BW, DP), jnp.float32) for _ in range(NBUF)],
        [pltpu.VMEM((D, BW), jnp.float32) for _ in range(NOBUF)],
        pltpu.VMEM((L, DP), jnp.float32),
        [pltpu.SemaphoreType.DMA for _ in range(NBUF)],
        [pltpu.SemaphoreType.DMA for _ in range(NOBUF)],
    ],
    compiler_params=_PARAMS,
)
def _emb_kernel(seq_t_hbm, w_hbm, pe_hbm, out_hbm,
                idx_bufs, rows_bufs, obufs, pe_v, gsems, osems):
    wid = lax.axis_index("s") * NC + lax.axis_index("c")
    b0 = wid * BW
    pltpu.sync_copy(pe_hbm, pe_v)

    iota = jax.lax.iota(jnp.int32, 16)

    def fire_gather(g, p):
        pltpu.sync_copy(
            seq_t_hbm.at[pl.ds(g, 1), pl.ds(b0, BW)], idx_bufs[p]
        )
        pltpu.async_copy(
            w_hbm.at[idx_bufs[p].at[0]], rows_bufs[p], gsems[p]
        )

    def wait_gather(p):
        pltpu.make_async_copy(
            w_hbm.at[pl.ds(0, BW)], rows_bufs[p], gsems[p]
        ).wait()

    def wait_out(q):
        pltpu.make_async_copy(
            obufs[q], out_hbm.at[0, :, pl.ds(0, BW)], osems[q]
        ).wait()

    fire_gather(0, 0)

    def outer(h, carry):
        for pp in range(NBUF):
            g = h * NBUF + pp
            q = pp % NOBUF
            p2 = (pp + 1) % NBUF

            @pl.when(g + 1 < L)
            def _():
                fire_gather(g + 1, p2)

            wait_gather(pp)

            @pl.when(g >= NOBUF)
            def _():
                wait_out(q)

            rows = rows_bufs[pp]
            ob = obufs[q]
            _transpose_diag(rows, ob, iota, BW // 16, D // 16,
                            pe=pe_v, g=g)

            pltpu.async_copy(
                ob, out_hbm.at[g, :, pl.ds(b0, BW)], osems[q]
            )
        return carry

    lax.fori_loop(0, L // NBUF, outer, 0)

    for q in range(NOBUF):
        wait_out(q)


def kernel(seq, weight):
    tail = VOCAB - NTFULL * DP
    wt_tail = jnp.pad(weight[NTFULL * DP:].T, ((0, 0), (0, DP - tail)))
    tab = _relayout_kernel(weight.T, wt_tail)
    kout = _emb_kernel(seq.T, tab, _pe_pad())
    return jnp.transpose(kout, (2, 0, 1))
